# baseline (speedup 1.0000x reference)
import functools

import jax
import jax.numpy as jnp
from jax import lax
from jax.experimental import pallas as pl
from jax.experimental.pallas import tpu as pltpu
from jax.experimental.pallas import tpu_sc as plsc

N = 2048
E = 32768
D = 256
H = 256

NC = 2
NS = 16

EROWS = E // 128
SROWS = 32
SCELLS = SROWS * N
NPASS = N // (NC * NS * SROWS)
CH = 32
CHUNKS = E // CH
CPW = CHUNKS // (NC * NS)
UROWS = EROWS // 32


def _sc_fused_body(flat_ref, p_ref, q_ref, nt_ref, row_ref, col_ref,
                   adj_ref, et_ref, g_ref,
                   fcv, slab, ridx, cidx, pbuf, qbuf, tbuf,
                   sem_p, sem_q, sem_t, sem_w):
    c = lax.axis_index("c")
    s = lax.axis_index("s")
    w = s * NC + c

    pltpu.sync_copy(flat_ref, fcv)

    def _zero_slab():
        def _z(i, _):
            for k in range(8):
                slab[pl.ds(i * 128 + k * 16, 16)] = jnp.zeros((16,), jnp.int32)
            return 0
        lax.fori_loop(0, SCELLS // 128, _z, 0)

    def _sweep_unit2(p, su):
        base = ((w * NPASS + p) * SROWS) * N

        def _sw(i, _):
            for k in range(8):
                f = fcv[i, pl.ds(k * 16, 16)]
                cell = f - base
                ok = (cell >= 0) & (cell < SCELLS)
                one = lax.shift_right_arithmetic(cell, 31) + 1
                plsc.store_scatter(slab, [cell], one, mask=ok)
            return 0
        lax.fori_loop(su * UROWS, (su + 2) * UROWS, _sw, 0)

    def _writeback(p):
        base = ((w * NPASS + p) * SROWS) * N
        pltpu.sync_copy(slab, adj_ref.at[pl.ds(base, SCELLS)])

    _zero_slab()
    prev_we = None
    prev_wg = None
    for t in range(CPW):
        ch = w * CPW + t
        pltpu.sync_copy(row_ref.at[ch], ridx)
        pltpu.sync_copy(col_ref.at[ch], cidx)
        cp = pltpu.async_copy(p_ref.at[ridx], pbuf, sem_p)
        cq = pltpu.async_copy(q_ref.at[cidx], qbuf, sem_q)
        ct = pltpu.async_copy(nt_ref.at[cidx], tbuf, sem_t)

        u0 = 2 * t
        _sweep_unit2(u0 // 32, u0 % 32)
        if u0 + 1 == 31:
            _writeback(0)
            _zero_slab()

        cp.wait()
        cq.wait()

        def _addg(i, _):
            e = lax.shift_right_logical(i, 4)
            goff = (i & 15) * 16
            pbuf[e, pl.ds(goff, 16)] = (
                pbuf[e, pl.ds(goff, 16)] + qbuf[e, pl.ds(goff, 16)])
            return 0
        lax.fori_loop(0, CH * (H // 16), _addg, 0)
        ct.wait()

        if prev_we is not None:
            prev_we.wait()
            prev_wg.wait()
        prev_we = pltpu.async_copy(pbuf, et_ref.at[pl.ds(ch * CH, CH)], sem_w)
        prev_wg = pltpu.async_copy(tbuf, g_ref.at[pl.ds(ch * CH, CH)], sem_w)
        prev_we.wait()
        prev_wg.wait()
        prev_we = None
        prev_wg = None
    _writeback(1)


def _sc_mesh():
    return plsc.VectorSubcoreMesh(core_axis_name="c", subcore_axis_name="s",
                                  num_cores=NC, num_subcores=NS)


def _sc_fused(flat2, P, Q, nt, row2, col2):
    return pl.kernel(
        _sc_fused_body,
        out_type=[jax.ShapeDtypeStruct((N * N,), jnp.int32),
                  jax.ShapeDtypeStruct((E, H), jnp.float32),
                  jax.ShapeDtypeStruct((E, H), jnp.float32)],
        mesh=_sc_mesh(),
        compiler_params=pltpu.CompilerParams(needs_layout_passes=False),
        scratch_types=[
            pltpu.VMEM((EROWS, 128), jnp.int32),
            pltpu.VMEM((SCELLS,), jnp.int32),
            pltpu.VMEM((CH,), jnp.int32),
            pltpu.VMEM((CH,), jnp.int32),
            pltpu.VMEM((CH, H), jnp.float32),
            pltpu.VMEM((CH, H), jnp.float32),
            pltpu.VMEM((CH, H), jnp.float32),
            pltpu.SemaphoreType.DMA,
            pltpu.SemaphoreType.DMA,
            pltpu.SemaphoreType.DMA,
            pltpu.SemaphoreType.DMA,
        ],
    )(flat2, P, Q, nt, row2, col2)


ECH = 4096
ESTEPS = E // ECH


def _tc_agg_body(row_ref, g_ref, nsum_ref, deg_ref):
    e = pl.program_id(0)
    rowv = row_ref[...].reshape(1, ECH)
    iota_i = lax.broadcasted_iota(jnp.int32, (N, ECH), 0)
    oht = (iota_i == rowv).astype(jnp.bfloat16)
    g = g_ref[...].astype(jnp.bfloat16)
    part = jnp.dot(oht, g, preferred_element_type=jnp.float32)
    dpart = jnp.sum(oht.astype(jnp.float32), axis=1, keepdims=True)
    dpart = jnp.broadcast_to(dpart, (N, 128))

    @pl.when(e == 0)
    def _init():
        nsum_ref[...] = part
        deg_ref[...] = dpart

    @pl.when(e > 0)
    def _acc():
        nsum_ref[...] = nsum_ref[...] + part
        deg_ref[...] = deg_ref[...] + dpart


def _tc_agg(rowE, G):
    return pl.pallas_call(
        _tc_agg_body,
        grid=(ESTEPS,),
        in_specs=[
            pl.BlockSpec((1, 1, ECH), lambda e: (e, 0, 0)),
            pl.BlockSpec((ECH, H), lambda e: (e, 0)),
        ],
        out_specs=[
            pl.BlockSpec((N, H), lambda e: (0, 0)),
            pl.BlockSpec((N, 128), lambda e: (0, 0)),
        ],
        out_shape=[
            jax.ShapeDtypeStruct((N, H), jnp.float32),
            jax.ShapeDtypeStruct((N, 128), jnp.float32),
        ],
    )(rowE, G)


def _tc_prep_body(x_ref, wn_ref, bn_ref, we_ref, be_ref, row_ref, col_ref,
                  nt_ref, p_ref, q_ref, flat_ref):
    xv = x_ref[...]
    nt_ref[...] = jnp.dot(xv, wn_ref[...],
                          preferred_element_type=jnp.float32) + bn_ref[...]
    p_ref[...] = jnp.dot(xv, we_ref[:D, :],
                         preferred_element_type=jnp.float32) + be_ref[...]
    q_ref[...] = jnp.dot(xv, we_ref[D:, :], preferred_element_type=jnp.float32)
    flat_ref[...] = row_ref[...] * N + col_ref[...]


def _tc_prep(x, Wn, bn, We, be, row2, col2):
    return pl.pallas_call(
        _tc_prep_body,
        out_shape=[
            jax.ShapeDtypeStruct((N, H), jnp.float32),
            jax.ShapeDtypeStruct((N, H), jnp.float32),
            jax.ShapeDtypeStruct((N, H), jnp.float32),
            jax.ShapeDtypeStruct((EROWS, 128), jnp.int32),
        ],
    )(x, Wn, bn, We, be, row2, col2)


BLK = 256


def _tc_main_body(ab_ref, af_ref, ntf_ref, ntb_ref, deg_ref, nsum_ref,
                  ws1_ref, bs1_ref, ws2_ref, bs2_ref,
                  wh1_ref, bh1_ref, wh2_ref, bh2_ref,
                  sub_ref, nb_ref):
    i = pl.program_id(0)
    ntf = ntf_ref[...]
    ntb = ntb_ref[...]

    deg = deg_ref[...][:, :1]
    nsum = nsum_ref[...]
    mean1 = jnp.where(deg > 0, nsum / jnp.maximum(deg, 1.0), 0.0)

    a2 = jnp.dot(ab_ref[...], af_ref[...], preferred_element_type=jnp.float32)
    rid = i * BLK + lax.broadcasted_iota(jnp.int32, (BLK, N), 0)
    cid = lax.broadcasted_iota(jnp.int32, (BLK, N), 1)
    m2 = jnp.where((a2 > 0) & (rid != cid), 1.0, 0.0)
    cnt2 = jnp.sum(m2, axis=1, keepdims=True)
    sum2 = jnp.dot(m2, ntf, preferred_element_type=jnp.float32)
    mean2 = jnp.where(cnt2 > 0, sum2 / jnp.maximum(cnt2, 1.0), 0.0)

    hs = jnp.maximum(
        jnp.dot(ntb, ws1_ref[:H, :], preferred_element_type=jnp.float32)
        + jnp.dot(mean1, ws1_ref[H:, :], preferred_element_type=jnp.float32)
        + bs1_ref[...], 0.0)
    sub_ref[...] = jnp.dot(hs, ws2_ref[...], preferred_element_type=jnp.float32,
                           ) + bs2_ref[...]

    hn = jnp.maximum(
        jnp.dot(ntb, wh1_ref[:H, :], preferred_element_type=jnp.float32)
        + jnp.dot(mean1, wh1_ref[H:2 * H, :], preferred_element_type=jnp.float32)
        + jnp.dot(mean2, wh1_ref[2 * H:, :], preferred_element_type=jnp.float32)
        + bh1_ref[...], 0.0)
    nb_ref[...] = jnp.dot(hn, wh2_ref[...], preferred_element_type=jnp.float32,
                          ) + bh2_ref[...]


def _tc_main(Ab, nt, deg, nsum, Ws1, bs1, Ws2, bs2, Wh1, bh1, Wh2, bh2):
    return pl.pallas_call(
        _tc_main_body,
        grid=(N // BLK,),
        in_specs=[
            pl.BlockSpec((BLK, N), lambda i: (i, 0)),
            pl.BlockSpec((N, N), lambda i: (0, 0)),
            pl.BlockSpec((N, H), lambda i: (0, 0)),
            pl.BlockSpec((BLK, H), lambda i: (i, 0)),
            pl.BlockSpec((BLK, 128), lambda i: (i, 0)),
            pl.BlockSpec((BLK, H), lambda i: (i, 0)),
            pl.BlockSpec((2 * H, H), lambda i: (0, 0)),
            pl.BlockSpec((1, H), lambda i: (0, 0)),
            pl.BlockSpec((H, H), lambda i: (0, 0)),
            pl.BlockSpec((1, H), lambda i: (0, 0)),
            pl.BlockSpec((3 * H, H), lambda i: (0, 0)),
            pl.BlockSpec((1, H), lambda i: (0, 0)),
            pl.BlockSpec((H, H), lambda i: (0, 0)),
            pl.BlockSpec((1, H), lambda i: (0, 0)),
        ],
        out_specs=[
            pl.BlockSpec((BLK, H), lambda i: (i, 0)),
            pl.BlockSpec((BLK, H), lambda i: (i, 0)),
        ],
        out_shape=[
            jax.ShapeDtypeStruct((N, H), jnp.float32),
            jax.ShapeDtypeStruct((N, H), jnp.float32),
        ],
    )(Ab, Ab, nt, nt, deg, nsum, Ws1, bs1, Ws2, bs2, Wh1, bh1, Wh2, bh2)


def kernel(x, edge_index, Wn, bn, We, be, Ws1, bs1, Ws2, bs2, Wh1, bh1, Wh2, bh2):
    row = edge_index[0]
    col = edge_index[1]

    nt, P, Q, flat2 = _tc_prep(x, Wn, bn.reshape(1, H), We, be.reshape(1, H),
                               row.reshape(EROWS, 128), col.reshape(EROWS, 128))

    A, edge_tokens, G = _sc_fused(flat2, P, Q, nt, row.reshape(CHUNKS, CH),
                                  col.reshape(CHUNKS, CH))
    Ab = A.reshape(N, N).astype(jnp.bfloat16)

    nsum, deg = _tc_agg(row.reshape(ESTEPS, 1, ECH), G)

    sub, nb = _tc_main(Ab, nt, deg, nsum, Ws1, bs1.reshape(1, H),
                       Ws2, bs2.reshape(1, H), Wh1, bh1.reshape(1, H),
                       Wh2, bh2.reshape(1, H))
    return nt, edge_tokens, sub, nb


# speedup vs baseline: 1.2336x; 1.2336x over previous
import functools

import jax
import jax.numpy as jnp
from jax import lax
from jax.experimental import pallas as pl
from jax.experimental.pallas import tpu as pltpu
from jax.experimental.pallas import tpu_sc as plsc

N = 2048
E = 32768
D = 256
H = 256

NC = 2
NS = 16

EROWS = E // 128
SROWS = 32
SCELLS = SROWS * N
NPASS = N // (NC * NS * SROWS)


def _sc_adj_body(flat_ref, out_ref, fcv, slab):
    c = lax.axis_index("c")
    s = lax.axis_index("s")
    w = s * NC + c

    pltpu.sync_copy(flat_ref, fcv)

    for p in range(NPASS):
        rowbase = (w * NPASS + p) * SROWS
        base = rowbase * N

        def _zeroall(i, _):
            for k in range(8):
                slab[pl.ds(i * 128 + k * 16, 16)] = jnp.zeros((16,), jnp.int32)
            return 0
        lax.fori_loop(0, SCELLS // 128, _zeroall, 0)

        def _sweep(i, _):
            for k in range(8):
                f = fcv[i, pl.ds(k * 16, 16)]
                cell = f - base
                ok = (cell >= 0) & (cell < SCELLS)
                one = lax.shift_right_arithmetic(cell, 31) + 1
                plsc.store_scatter(slab, [cell], one, mask=ok)
            return 0
        lax.fori_loop(0, EROWS, _sweep, 0)

        pltpu.sync_copy(slab, out_ref.at[pl.ds(base, SCELLS)])


def _sc_mesh():
    return plsc.VectorSubcoreMesh(core_axis_name="c", subcore_axis_name="s",
                                  num_cores=NC, num_subcores=NS)


def _sc_adj(flat2):
    return pl.kernel(
        _sc_adj_body,
        out_type=jax.ShapeDtypeStruct((N * N,), jnp.int32),
        mesh=_sc_mesh(),
        compiler_params=pltpu.CompilerParams(needs_layout_passes=False),
        scratch_types=[
            pltpu.VMEM((EROWS, 128), jnp.int32),
            pltpu.VMEM((SCELLS,), jnp.int32),
        ],
    )(flat2)


CH = 64
CHUNKS = E // CH
CPW = CHUNKS // (NC * NS)


def _sc_edges_body(p_ref, q_ref, nt_ref, row_ref, col_ref, out_ref, g_ref,
                   ridx0, cidx0, ridx1, cidx1, pbuf0, qbuf0, tbuf0,
                   pbuf1, qbuf1, tbuf1, sem_p, sem_q, sem_t, sem_w):
    c = lax.axis_index("c")
    s = lax.axis_index("s")
    w = s * NC + c
    bufs = ((ridx0, cidx0, pbuf0, qbuf0, tbuf0),
            (ridx1, cidx1, pbuf1, qbuf1, tbuf1))

    def _fire(t, b):
        ridx, cidx, pbuf, qbuf, tbuf = bufs[b]
        ch = w * CPW + t
        pltpu.sync_copy(row_ref.at[ch], ridx)
        pltpu.sync_copy(col_ref.at[ch], cidx)
        return (pltpu.async_copy(p_ref.at[ridx], pbuf, sem_p),
                pltpu.async_copy(q_ref.at[cidx], qbuf, sem_q),
                pltpu.async_copy(nt_ref.at[cidx], tbuf, sem_t))

    def _consume(t, b, cps):
        ridx, cidx, pbuf, qbuf, tbuf = bufs[b]
        ch = w * CPW + t
        cps[0].wait()
        cps[1].wait()

        def _addg(i, _):
            e = lax.shift_right_logical(i, 4)
            goff = (i & 15) * 16
            pbuf[e, pl.ds(goff, 16)] = (
                pbuf[e, pl.ds(goff, 16)] + qbuf[e, pl.ds(goff, 16)])
            return 0
        lax.fori_loop(0, CH * (H // 16), _addg, 0)
        cps[2].wait()
        we = pltpu.async_copy(pbuf, out_ref.at[pl.ds(ch * CH, CH)], sem_w)
        wg = pltpu.async_copy(tbuf, g_ref.at[pl.ds(ch * CH, CH)], sem_w)
        return we, wg

    pend = _fire(0, 0)
    pw = None
    for t in range(CPW):
        b = t % 2
        nxt = _fire(t + 1, 1 - b) if t + 1 < CPW else None
        if pw is not None:
            pw[0].wait()
            pw[1].wait()
        pw = _consume(t, b, pend)
        pend = nxt
    pw[0].wait()
    pw[1].wait()


def _sc_edges(P, Q, nt, row2, col2):
    return pl.kernel(
        _sc_edges_body,
        out_type=[jax.ShapeDtypeStruct((E, H), jnp.float32),
                  jax.ShapeDtypeStruct((E, H), jnp.float32)],
        mesh=_sc_mesh(),
        scratch_types=[
            pltpu.VMEM((CH,), jnp.int32),
            pltpu.VMEM((CH,), jnp.int32),
            pltpu.VMEM((CH,), jnp.int32),
            pltpu.VMEM((CH,), jnp.int32),
            pltpu.VMEM((CH, H), jnp.float32),
            pltpu.VMEM((CH, H), jnp.float32),
            pltpu.VMEM((CH, H), jnp.float32),
            pltpu.VMEM((CH, H), jnp.float32),
            pltpu.VMEM((CH, H), jnp.float32),
            pltpu.VMEM((CH, H), jnp.float32),
            pltpu.SemaphoreType.DMA,
            pltpu.SemaphoreType.DMA,
            pltpu.SemaphoreType.DMA,
            pltpu.SemaphoreType.DMA,
        ],
    )(P, Q, nt, row2, col2)


ECH = 4096
ESTEPS = E // ECH


def _tc_agg_body(row_ref, g_ref, nsum_ref, deg_ref):
    e = pl.program_id(0)
    rowv = row_ref[...].reshape(1, ECH)
    iota_i = lax.broadcasted_iota(jnp.int32, (N, ECH), 0)
    oht = (iota_i == rowv).astype(jnp.bfloat16)
    g = g_ref[...].astype(jnp.bfloat16)
    part = jnp.dot(oht, g, preferred_element_type=jnp.float32)
    dpart = jnp.sum(oht.astype(jnp.float32), axis=1, keepdims=True)
    dpart = jnp.broadcast_to(dpart, (N, 128))

    @pl.when(e == 0)
    def _init():
        nsum_ref[...] = part
        deg_ref[...] = dpart

    @pl.when(e > 0)
    def _acc():
        nsum_ref[...] = nsum_ref[...] + part
        deg_ref[...] = deg_ref[...] + dpart


def _tc_agg(rowE, G):
    return pl.pallas_call(
        _tc_agg_body,
        grid=(ESTEPS,),
        in_specs=[
            pl.BlockSpec((1, 1, ECH), lambda e: (e, 0, 0)),
            pl.BlockSpec((ECH, H), lambda e: (e, 0)),
        ],
        out_specs=[
            pl.BlockSpec((N, H), lambda e: (0, 0)),
            pl.BlockSpec((N, 128), lambda e: (0, 0)),
        ],
        out_shape=[
            jax.ShapeDtypeStruct((N, H), jnp.float32),
            jax.ShapeDtypeStruct((N, 128), jnp.float32),
        ],
    )(rowE, G)


def _tc_prep_body(x_ref, wn_ref, bn_ref, we_ref, be_ref, row_ref, col_ref,
                  nt_ref, p_ref, q_ref, flat_ref):
    xv = x_ref[...]
    nt_ref[...] = jnp.dot(xv, wn_ref[...],
                          preferred_element_type=jnp.float32) + bn_ref[...]
    p_ref[...] = jnp.dot(xv, we_ref[:D, :],
                         preferred_element_type=jnp.float32) + be_ref[...]
    q_ref[...] = jnp.dot(xv, we_ref[D:, :], preferred_element_type=jnp.float32)
    flat_ref[...] = row_ref[...] * N + col_ref[...]


def _tc_prep(x, Wn, bn, We, be, row2, col2):
    return pl.pallas_call(
        _tc_prep_body,
        out_shape=[
            jax.ShapeDtypeStruct((N, H), jnp.float32),
            jax.ShapeDtypeStruct((N, H), jnp.float32),
            jax.ShapeDtypeStruct((N, H), jnp.float32),
            jax.ShapeDtypeStruct((EROWS, 128), jnp.int32),
        ],
    )(x, Wn, bn, We, be, row2, col2)


BLK = 256


def _tc_main_body(ab_ref, af_ref, ntf_ref, ntb_ref, deg_ref, nsum_ref,
                  ws1_ref, bs1_ref, ws2_ref, bs2_ref,
                  wh1_ref, bh1_ref, wh2_ref, bh2_ref,
                  sub_ref, nb_ref):
    i = pl.program_id(0)
    ntf = ntf_ref[...]
    ntb = ntb_ref[...]

    deg = deg_ref[...][:, :1]
    nsum = nsum_ref[...]
    mean1 = jnp.where(deg > 0, nsum / jnp.maximum(deg, 1.0), 0.0)

    a2 = jnp.dot(ab_ref[...], af_ref[...], preferred_element_type=jnp.float32)
    rid = i * BLK + lax.broadcasted_iota(jnp.int32, (BLK, N), 0)
    cid = lax.broadcasted_iota(jnp.int32, (BLK, N), 1)
    m2 = jnp.where((a2 > 0) & (rid != cid), 1.0, 0.0)
    cnt2 = jnp.sum(m2, axis=1, keepdims=True)
    sum2 = jnp.dot(m2, ntf, preferred_element_type=jnp.float32)
    mean2 = jnp.where(cnt2 > 0, sum2 / jnp.maximum(cnt2, 1.0), 0.0)

    hs = jnp.maximum(
        jnp.dot(ntb, ws1_ref[:H, :], preferred_element_type=jnp.float32)
        + jnp.dot(mean1, ws1_ref[H:, :], preferred_element_type=jnp.float32)
        + bs1_ref[...], 0.0)
    sub_ref[...] = jnp.dot(hs, ws2_ref[...], preferred_element_type=jnp.float32,
                           ) + bs2_ref[...]

    hn = jnp.maximum(
        jnp.dot(ntb, wh1_ref[:H, :], preferred_element_type=jnp.float32)
        + jnp.dot(mean1, wh1_ref[H:2 * H, :], preferred_element_type=jnp.float32)
        + jnp.dot(mean2, wh1_ref[2 * H:, :], preferred_element_type=jnp.float32)
        + bh1_ref[...], 0.0)
    nb_ref[...] = jnp.dot(hn, wh2_ref[...], preferred_element_type=jnp.float32,
                          ) + bh2_ref[...]


def _tc_main(Ab, nt, deg, nsum, Ws1, bs1, Ws2, bs2, Wh1, bh1, Wh2, bh2):
    return pl.pallas_call(
        _tc_main_body,
        grid=(N // BLK,),
        in_specs=[
            pl.BlockSpec((BLK, N), lambda i: (i, 0)),
            pl.BlockSpec((N, N), lambda i: (0, 0)),
            pl.BlockSpec((N, H), lambda i: (0, 0)),
            pl.BlockSpec((BLK, H), lambda i: (i, 0)),
            pl.BlockSpec((BLK, 128), lambda i: (i, 0)),
            pl.BlockSpec((BLK, H), lambda i: (i, 0)),
            pl.BlockSpec((2 * H, H), lambda i: (0, 0)),
            pl.BlockSpec((1, H), lambda i: (0, 0)),
            pl.BlockSpec((H, H), lambda i: (0, 0)),
            pl.BlockSpec((1, H), lambda i: (0, 0)),
            pl.BlockSpec((3 * H, H), lambda i: (0, 0)),
            pl.BlockSpec((1, H), lambda i: (0, 0)),
            pl.BlockSpec((H, H), lambda i: (0, 0)),
            pl.BlockSpec((1, H), lambda i: (0, 0)),
        ],
        out_specs=[
            pl.BlockSpec((BLK, H), lambda i: (i, 0)),
            pl.BlockSpec((BLK, H), lambda i: (i, 0)),
        ],
        out_shape=[
            jax.ShapeDtypeStruct((N, H), jnp.float32),
            jax.ShapeDtypeStruct((N, H), jnp.float32),
        ],
    )(Ab, Ab, nt, nt, deg, nsum, Ws1, bs1, Ws2, bs2, Wh1, bh1, Wh2, bh2)


def kernel(x, edge_index, Wn, bn, We, be, Ws1, bs1, Ws2, bs2, Wh1, bh1, Wh2, bh2):
    row = edge_index[0]
    col = edge_index[1]

    nt, P, Q, flat2 = _tc_prep(x, Wn, bn.reshape(1, H), We, be.reshape(1, H),
                               row.reshape(EROWS, 128), col.reshape(EROWS, 128))

    Ab = _sc_adj(flat2).reshape(N, N).astype(jnp.bfloat16)

    edge_tokens, G = _sc_edges(P, Q, nt, row.reshape(CHUNKS, CH),
                               col.reshape(CHUNKS, CH))

    nsum, deg = _tc_agg(row.reshape(ESTEPS, 1, ECH), G)

    sub, nb = _tc_main(Ab, nt, deg, nsum, Ws1, bs1.reshape(1, H),
                       Ws2, bs2.reshape(1, H), Wh1, bh1.reshape(1, H),
                       Wh2, bh2.reshape(1, H))
    return nt, edge_tokens, sub, nb


# speedup vs baseline: 1.5004x; 1.2163x over previous
import functools

import jax
import jax.numpy as jnp
from jax import lax
from jax.experimental import pallas as pl
from jax.experimental.pallas import tpu as pltpu
from jax.experimental.pallas import tpu_sc as plsc

N = 2048
E = 32768
D = 256
H = 256

NC = 2
NS = 16

EROWS = E // 128
SROWS = 32
SCELLS = SROWS * N
NPASS = N // (NC * NS * SROWS)


def _sc_adj_body(flat_ref, out_ref, fcv, slab):
    c = lax.axis_index("c")
    s = lax.axis_index("s")
    w = s * NC + c

    pltpu.sync_copy(flat_ref, fcv)

    for p in range(NPASS):
        rowbase = (w * NPASS + p) * SROWS
        base = rowbase * N

        def _zeroall(i, _):
            for k in range(8):
                slab[pl.ds(i * 128 + k * 16, 16)] = jnp.zeros((16,), jnp.int32)
            return 0
        lax.fori_loop(0, SCELLS // 128, _zeroall, 0)

        def _sweep(i, _):
            for k in range(8):
                f = fcv[i, pl.ds(k * 16, 16)]
                cell = f - base
                ok = (cell >= 0) & (cell < SCELLS)
                one = lax.shift_right_arithmetic(cell, 31) + 1
                plsc.store_scatter(slab, [cell], one, mask=ok)
            return 0
        lax.fori_loop(0, EROWS, _sweep, 0)

        pltpu.sync_copy(slab, out_ref.at[pl.ds(base, SCELLS)])


def _sc_mesh():
    return plsc.VectorSubcoreMesh(core_axis_name="c", subcore_axis_name="s",
                                  num_cores=NC, num_subcores=NS)


def _sc_adj(flat2):
    return pl.kernel(
        _sc_adj_body,
        out_type=jax.ShapeDtypeStruct((N * N,), jnp.int32),
        mesh=_sc_mesh(),
        compiler_params=pltpu.CompilerParams(needs_layout_passes=False),
        scratch_types=[
            pltpu.VMEM((EROWS, 128), jnp.int32),
            pltpu.VMEM((SCELLS,), jnp.int32),
        ],
    )(flat2)


CH = 64
CHUNKS = E // CH
CPW = CHUNKS // (NC * NS)


def _sc_edges_body(p_ref, q_ref, row_ref, col_ref, out_ref,
                   ridx0, cidx0, ridx1, cidx1, pbuf0, qbuf0,
                   pbuf1, qbuf1, sem_p, sem_q, sem_w):
    c = lax.axis_index("c")
    s = lax.axis_index("s")
    w = s * NC + c
    bufs = ((ridx0, cidx0, pbuf0, qbuf0),
            (ridx1, cidx1, pbuf1, qbuf1))

    def _fire(t, b):
        ridx, cidx, pbuf, qbuf = bufs[b]
        ch = w * CPW + t
        pltpu.sync_copy(row_ref.at[ch], ridx)
        pltpu.sync_copy(col_ref.at[ch], cidx)
        return (pltpu.async_copy(p_ref.at[ridx], pbuf, sem_p),
                pltpu.async_copy(q_ref.at[cidx], qbuf, sem_q))

    def _consume(t, b, cps):
        ridx, cidx, pbuf, qbuf = bufs[b]
        ch = w * CPW + t
        cps[0].wait()
        cps[1].wait()

        def _addg(i, _):
            e = lax.shift_right_logical(i, 4)
            goff = (i & 15) * 16
            pbuf[e, pl.ds(goff, 16)] = (
                pbuf[e, pl.ds(goff, 16)] + qbuf[e, pl.ds(goff, 16)])
            return 0
        lax.fori_loop(0, CH * (H // 16), _addg, 0)
        return pltpu.async_copy(pbuf, out_ref.at[pl.ds(ch * CH, CH)], sem_w)

    pend = _fire(0, 0)
    pw = None
    for t in range(CPW):
        b = t % 2
        nxt = _fire(t + 1, 1 - b) if t + 1 < CPW else None
        if pw is not None:
            pw.wait()
        pw = _consume(t, b, pend)
        pend = nxt
    pw.wait()


def _sc_edges(P, Q, row2, col2):
    return pl.kernel(
        _sc_edges_body,
        out_type=jax.ShapeDtypeStruct((E, H), jnp.float32),
        mesh=_sc_mesh(),
        scratch_types=[
            pltpu.VMEM((CH,), jnp.int32),
            pltpu.VMEM((CH,), jnp.int32),
            pltpu.VMEM((CH,), jnp.int32),
            pltpu.VMEM((CH,), jnp.int32),
            pltpu.VMEM((CH, H), jnp.float32),
            pltpu.VMEM((CH, H), jnp.float32),
            pltpu.VMEM((CH, H), jnp.float32),
            pltpu.VMEM((CH, H), jnp.float32),
            pltpu.SemaphoreType.DMA,
            pltpu.SemaphoreType.DMA,
            pltpu.SemaphoreType.DMA,
        ],
    )(P, Q, row2, col2)


ECH = 4096
ESTEPS = E // ECH


def _tc_agg_body(row_ref, col_ref, ntb_ref, nsum_ref, deg_ref):
    e = pl.program_id(0)
    rowv = row_ref[...].reshape(1, ECH)
    colv = col_ref[...].reshape(ECH, 1)
    iota_i = lax.broadcasted_iota(jnp.int32, (N, ECH), 0)
    ohr = (iota_i == rowv).astype(jnp.bfloat16)
    iota_n = lax.broadcasted_iota(jnp.int32, (ECH, N), 1)
    ohc = (iota_n == colv).astype(jnp.bfloat16)
    gch = jnp.dot(ohc, ntb_ref[...], preferred_element_type=jnp.float32).astype(jnp.bfloat16)
    part = jnp.dot(ohr, gch, preferred_element_type=jnp.float32)
    dpart = jnp.sum(ohr.astype(jnp.float32), axis=1, keepdims=True)
    dpart = jnp.broadcast_to(dpart, (N, 128))

    @pl.when(e == 0)
    def _init():
        nsum_ref[...] = part
        deg_ref[...] = dpart

    @pl.when(e > 0)
    def _acc():
        nsum_ref[...] = nsum_ref[...] + part
        deg_ref[...] = deg_ref[...] + dpart


def _tc_agg(rowE, colE, ntb):
    return pl.pallas_call(
        _tc_agg_body,
        grid=(ESTEPS,),
        in_specs=[
            pl.BlockSpec((1, 1, ECH), lambda e: (e, 0, 0)),
            pl.BlockSpec((1, 1, ECH), lambda e: (e, 0, 0)),
            pl.BlockSpec((N, H), lambda e: (0, 0)),
        ],
        out_specs=[
            pl.BlockSpec((N, H), lambda e: (0, 0)),
            pl.BlockSpec((N, 128), lambda e: (0, 0)),
        ],
        out_shape=[
            jax.ShapeDtypeStruct((N, H), jnp.float32),
            jax.ShapeDtypeStruct((N, 128), jnp.float32),
        ],
    )(rowE, colE, ntb)


def _tc_prep_body(x_ref, wn_ref, bn_ref, we_ref, be_ref, row_ref, col_ref,
                  nt_ref, p_ref, q_ref, flat_ref):
    xv = x_ref[...]
    nt_ref[...] = jnp.dot(xv, wn_ref[...],
                          preferred_element_type=jnp.float32) + bn_ref[...]
    p_ref[...] = jnp.dot(xv, we_ref[:D, :],
                         preferred_element_type=jnp.float32) + be_ref[...]
    q_ref[...] = jnp.dot(xv, we_ref[D:, :], preferred_element_type=jnp.float32)
    flat_ref[...] = row_ref[...] * N + col_ref[...]


def _tc_prep(x, Wn, bn, We, be, row2, col2):
    return pl.pallas_call(
        _tc_prep_body,
        out_shape=[
            jax.ShapeDtypeStruct((N, H), jnp.float32),
            jax.ShapeDtypeStruct((N, H), jnp.float32),
            jax.ShapeDtypeStruct((N, H), jnp.float32),
            jax.ShapeDtypeStruct((EROWS, 128), jnp.int32),
        ],
    )(x, Wn, bn, We, be, row2, col2)


BLK = 256


def _tc_main_body(ab_ref, af_ref, ntf_ref, ntb_ref, deg_ref, nsum_ref,
                  ws1_ref, bs1_ref, ws2_ref, bs2_ref,
                  wh1_ref, bh1_ref, wh2_ref, bh2_ref,
                  sub_ref, nb_ref):
    i = pl.program_id(0)
    ntf = ntf_ref[...]
    ntb = ntb_ref[...]

    deg = deg_ref[...][:, :1]
    nsum = nsum_ref[...]
    mean1 = jnp.where(deg > 0, nsum / jnp.maximum(deg, 1.0), 0.0)

    a2 = jnp.dot(ab_ref[...], af_ref[...], preferred_element_type=jnp.float32)
    rid = i * BLK + lax.broadcasted_iota(jnp.int32, (BLK, N), 0)
    cid = lax.broadcasted_iota(jnp.int32, (BLK, N), 1)
    m2 = jnp.where((a2 > 0) & (rid != cid), 1.0, 0.0)
    cnt2 = jnp.sum(m2, axis=1, keepdims=True)
    sum2 = jnp.dot(m2, ntf, preferred_element_type=jnp.float32)
    mean2 = jnp.where(cnt2 > 0, sum2 / jnp.maximum(cnt2, 1.0), 0.0)

    hs = jnp.maximum(
        jnp.dot(ntb, ws1_ref[:H, :], preferred_element_type=jnp.float32)
        + jnp.dot(mean1, ws1_ref[H:, :], preferred_element_type=jnp.float32)
        + bs1_ref[...], 0.0)
    sub_ref[...] = jnp.dot(hs, ws2_ref[...], preferred_element_type=jnp.float32,
                           ) + bs2_ref[...]

    hn = jnp.maximum(
        jnp.dot(ntb, wh1_ref[:H, :], preferred_element_type=jnp.float32)
        + jnp.dot(mean1, wh1_ref[H:2 * H, :], preferred_element_type=jnp.float32)
        + jnp.dot(mean2, wh1_ref[2 * H:, :], preferred_element_type=jnp.float32)
        + bh1_ref[...], 0.0)
    nb_ref[...] = jnp.dot(hn, wh2_ref[...], preferred_element_type=jnp.float32,
                          ) + bh2_ref[...]


def _tc_main(Ab, nt, deg, nsum, Ws1, bs1, Ws2, bs2, Wh1, bh1, Wh2, bh2):
    return pl.pallas_call(
        _tc_main_body,
        grid=(N // BLK,),
        in_specs=[
            pl.BlockSpec((BLK, N), lambda i: (i, 0)),
            pl.BlockSpec((N, N), lambda i: (0, 0)),
            pl.BlockSpec((N, H), lambda i: (0, 0)),
            pl.BlockSpec((BLK, H), lambda i: (i, 0)),
            pl.BlockSpec((BLK, 128), lambda i: (i, 0)),
            pl.BlockSpec((BLK, H), lambda i: (i, 0)),
            pl.BlockSpec((2 * H, H), lambda i: (0, 0)),
            pl.BlockSpec((1, H), lambda i: (0, 0)),
            pl.BlockSpec((H, H), lambda i: (0, 0)),
            pl.BlockSpec((1, H), lambda i: (0, 0)),
            pl.BlockSpec((3 * H, H), lambda i: (0, 0)),
            pl.BlockSpec((1, H), lambda i: (0, 0)),
            pl.BlockSpec((H, H), lambda i: (0, 0)),
            pl.BlockSpec((1, H), lambda i: (0, 0)),
        ],
        out_specs=[
            pl.BlockSpec((BLK, H), lambda i: (i, 0)),
            pl.BlockSpec((BLK, H), lambda i: (i, 0)),
        ],
        out_shape=[
            jax.ShapeDtypeStruct((N, H), jnp.float32),
            jax.ShapeDtypeStruct((N, H), jnp.float32),
        ],
    )(Ab, Ab, nt, nt, deg, nsum, Ws1, bs1, Ws2, bs2, Wh1, bh1, Wh2, bh2)


def kernel(x, edge_index, Wn, bn, We, be, Ws1, bs1, Ws2, bs2, Wh1, bh1, Wh2, bh2):
    row = edge_index[0]
    col = edge_index[1]

    nt, P, Q, flat2 = _tc_prep(x, Wn, bn.reshape(1, H), We, be.reshape(1, H),
                               row.reshape(EROWS, 128), col.reshape(EROWS, 128))

    Ab = _sc_adj(flat2).reshape(N, N).astype(jnp.bfloat16)

    edge_tokens = _sc_edges(P, Q, row.reshape(CHUNKS, CH),
                            col.reshape(CHUNKS, CH))

    nsum, deg = _tc_agg(row.reshape(ESTEPS, 1, ECH),
                        col.reshape(ESTEPS, 1, ECH),
                        nt.astype(jnp.bfloat16))

    sub, nb = _tc_main(Ab, nt, deg, nsum, Ws1, bs1.reshape(1, H),
                       Ws2, bs2.reshape(1, H), Wh1, bh1.reshape(1, H),
                       Wh2, bh2.reshape(1, H))
    return nt, edge_tokens, sub, nb


# speedup vs baseline: 1.5068x; 1.0043x over previous
import functools

import jax
import jax.numpy as jnp
from jax import lax
from jax.experimental import pallas as pl
from jax.experimental.pallas import tpu as pltpu
from jax.experimental.pallas import tpu_sc as plsc

N = 2048
E = 32768
D = 256
H = 256

NC = 2
NS = 16

EROWS = E // 128
SROWS = 32
SCELLS = SROWS * N
NPASS = N // (NC * NS * SROWS)


def _sc_adj_body(flat_ref, out_ref, fcv, slab):
    c = lax.axis_index("c")
    s = lax.axis_index("s")
    w = s * NC + c

    pltpu.sync_copy(flat_ref, fcv)

    for p in range(NPASS):
        rowbase = (w * NPASS + p) * SROWS
        base = rowbase * N

        def _zeroall(i, _):
            for k in range(8):
                slab[pl.ds(i * 128 + k * 16, 16)] = jnp.zeros((16,), jnp.int32)
            return 0
        lax.fori_loop(0, SCELLS // 128, _zeroall, 0)

        def _sweep(i, _):
            for k in range(8):
                f = fcv[i, pl.ds(k * 16, 16)]
                cell = f - base
                ok = (cell >= 0) & (cell < SCELLS)
                one = lax.shift_right_arithmetic(cell, 31) + 1
                plsc.store_scatter(slab, [cell], one, mask=ok)
            return 0
        lax.fori_loop(0, EROWS, _sweep, 0)

        pltpu.sync_copy(slab, out_ref.at[pl.ds(base, SCELLS)])


def _sc_mesh():
    return plsc.VectorSubcoreMesh(core_axis_name="c", subcore_axis_name="s",
                                  num_cores=NC, num_subcores=NS)


def _sc_adj(flat2):
    return pl.kernel(
        _sc_adj_body,
        out_type=jax.ShapeDtypeStruct((N * N,), jnp.int32),
        mesh=_sc_mesh(),
        compiler_params=pltpu.CompilerParams(needs_layout_passes=False),
        scratch_types=[
            pltpu.VMEM((EROWS, 128), jnp.int32),
            pltpu.VMEM((SCELLS,), jnp.int32),
        ],
    )(flat2)


CH = 64
CHUNKS = E // CH
CPW = CHUNKS // (NC * NS)


def _sc_edges_body(p_ref, q_ref, row_ref, col_ref, out_ref,
                   ridx, cidx, pbuf0, qbuf0, pbuf1, qbuf1,
                   sem_p, sem_q, sem_w):
    c = lax.axis_index("c")
    s = lax.axis_index("s")
    w = s * NC + c
    bufs = ((pbuf0, qbuf0), (pbuf1, qbuf1))

    pltpu.sync_copy(row_ref.at[pl.ds(w * CPW, CPW)], ridx)
    pltpu.sync_copy(col_ref.at[pl.ds(w * CPW, CPW)], cidx)

    def _fire(t, b):
        pbuf, qbuf = bufs[b]
        return (pltpu.async_copy(p_ref.at[ridx.at[t]], pbuf, sem_p),
                pltpu.async_copy(q_ref.at[cidx.at[t]], qbuf, sem_q))

    def _consume(t, b, cps):
        pbuf, qbuf = bufs[b]
        ch = w * CPW + t
        cps[0].wait()
        cps[1].wait()

        def _addg(i, _):
            e = lax.shift_right_logical(i, 4)
            goff = (i & 15) * 16
            pbuf[e, pl.ds(goff, 16)] = (
                pbuf[e, pl.ds(goff, 16)] + qbuf[e, pl.ds(goff, 16)])
            return 0
        lax.fori_loop(0, CH * (H // 16), _addg, 0)
        return pltpu.async_copy(pbuf, out_ref.at[pl.ds(ch * CH, CH)], sem_w)

    pend = _fire(0, 0)
    pw = None
    for t in range(CPW):
        b = t % 2
        nxt = _fire(t + 1, 1 - b) if t + 1 < CPW else None
        if pw is not None:
            pw.wait()
        pw = _consume(t, b, pend)
        pend = nxt
    pw.wait()


def _sc_edges(P, Q, row2, col2):
    return pl.kernel(
        _sc_edges_body,
        out_type=jax.ShapeDtypeStruct((E, H), jnp.float32),
        mesh=_sc_mesh(),
        scratch_types=[
            pltpu.VMEM((CPW, CH), jnp.int32),
            pltpu.VMEM((CPW, CH), jnp.int32),
            pltpu.VMEM((CH, H), jnp.float32),
            pltpu.VMEM((CH, H), jnp.float32),
            pltpu.VMEM((CH, H), jnp.float32),
            pltpu.VMEM((CH, H), jnp.float32),
            pltpu.SemaphoreType.DMA,
            pltpu.SemaphoreType.DMA,
            pltpu.SemaphoreType.DMA,
        ],
    )(P, Q, row2, col2)


ECH = 4096
ESTEPS = E // ECH


def _tc_agg_body(row_ref, col_ref, ntb_ref, nsum_ref, deg_ref):
    e = pl.program_id(0)
    rowv = row_ref[...].reshape(1, ECH)
    colv = col_ref[...].reshape(ECH, 1)
    iota_i = lax.broadcasted_iota(jnp.int32, (N, ECH), 0)
    ohr = (iota_i == rowv).astype(jnp.bfloat16)
    iota_n = lax.broadcasted_iota(jnp.int32, (ECH, N), 1)
    ohc = (iota_n == colv).astype(jnp.bfloat16)
    gch = jnp.dot(ohc, ntb_ref[...], preferred_element_type=jnp.float32).astype(jnp.bfloat16)
    part = jnp.dot(ohr, gch, preferred_element_type=jnp.float32)
    dpart = jnp.sum(ohr.astype(jnp.float32), axis=1, keepdims=True)
    dpart = jnp.broadcast_to(dpart, (N, 128))

    @pl.when(e == 0)
    def _init():
        nsum_ref[...] = part
        deg_ref[...] = dpart

    @pl.when(e > 0)
    def _acc():
        nsum_ref[...] = nsum_ref[...] + part
        deg_ref[...] = deg_ref[...] + dpart


def _tc_agg(rowE, colE, ntb):
    return pl.pallas_call(
        _tc_agg_body,
        grid=(ESTEPS,),
        in_specs=[
            pl.BlockSpec((1, 1, ECH), lambda e: (e, 0, 0)),
            pl.BlockSpec((1, 1, ECH), lambda e: (e, 0, 0)),
            pl.BlockSpec((N, H), lambda e: (0, 0)),
        ],
        out_specs=[
            pl.BlockSpec((N, H), lambda e: (0, 0)),
            pl.BlockSpec((N, 128), lambda e: (0, 0)),
        ],
        out_shape=[
            jax.ShapeDtypeStruct((N, H), jnp.float32),
            jax.ShapeDtypeStruct((N, 128), jnp.float32),
        ],
    )(rowE, colE, ntb)


def _tc_prep_body(x_ref, wn_ref, bn_ref, we_ref, be_ref, row_ref, col_ref,
                  nt_ref, p_ref, q_ref, flat_ref):
    xv = x_ref[...]
    nt_ref[...] = jnp.dot(xv, wn_ref[...],
                          preferred_element_type=jnp.float32) + bn_ref[...]
    p_ref[...] = jnp.dot(xv, we_ref[:D, :],
                         preferred_element_type=jnp.float32) + be_ref[...]
    q_ref[...] = jnp.dot(xv, we_ref[D:, :], preferred_element_type=jnp.float32)
    flat_ref[...] = row_ref[...] * N + col_ref[...]


def _tc_prep(x, Wn, bn, We, be, row2, col2):
    return pl.pallas_call(
        _tc_prep_body,
        out_shape=[
            jax.ShapeDtypeStruct((N, H), jnp.float32),
            jax.ShapeDtypeStruct((N, H), jnp.float32),
            jax.ShapeDtypeStruct((N, H), jnp.float32),
            jax.ShapeDtypeStruct((EROWS, 128), jnp.int32),
        ],
    )(x, Wn, bn, We, be, row2, col2)


BLK = 256


def _tc_main_body(ab_ref, af_ref, ntf_ref, ntb_ref, deg_ref, nsum_ref,
                  ws1_ref, bs1_ref, ws2_ref, bs2_ref,
                  wh1_ref, bh1_ref, wh2_ref, bh2_ref,
                  sub_ref, nb_ref):
    i = pl.program_id(0)
    ntf = ntf_ref[...]
    ntb = ntb_ref[...]

    deg = deg_ref[...][:, :1]
    nsum = nsum_ref[...]
    mean1 = jnp.where(deg > 0, nsum / jnp.maximum(deg, 1.0), 0.0)

    a2 = jnp.dot(ab_ref[...], af_ref[...], preferred_element_type=jnp.float32)
    rid = i * BLK + lax.broadcasted_iota(jnp.int32, (BLK, N), 0)
    cid = lax.broadcasted_iota(jnp.int32, (BLK, N), 1)
    m2 = jnp.where((a2 > 0) & (rid != cid), 1.0, 0.0)
    cnt2 = jnp.sum(m2, axis=1, keepdims=True)
    sum2 = jnp.dot(m2, ntf, preferred_element_type=jnp.float32)
    mean2 = jnp.where(cnt2 > 0, sum2 / jnp.maximum(cnt2, 1.0), 0.0)

    hs = jnp.maximum(
        jnp.dot(ntb, ws1_ref[:H, :], preferred_element_type=jnp.float32)
        + jnp.dot(mean1, ws1_ref[H:, :], preferred_element_type=jnp.float32)
        + bs1_ref[...], 0.0)
    sub_ref[...] = jnp.dot(hs, ws2_ref[...], preferred_element_type=jnp.float32,
                           ) + bs2_ref[...]

    hn = jnp.maximum(
        jnp.dot(ntb, wh1_ref[:H, :], preferred_element_type=jnp.float32)
        + jnp.dot(mean1, wh1_ref[H:2 * H, :], preferred_element_type=jnp.float32)
        + jnp.dot(mean2, wh1_ref[2 * H:, :], preferred_element_type=jnp.float32)
        + bh1_ref[...], 0.0)
    nb_ref[...] = jnp.dot(hn, wh2_ref[...], preferred_element_type=jnp.float32,
                          ) + bh2_ref[...]


def _tc_main(Ab, nt, deg, nsum, Ws1, bs1, Ws2, bs2, Wh1, bh1, Wh2, bh2):
    return pl.pallas_call(
        _tc_main_body,
        grid=(N // BLK,),
        in_specs=[
            pl.BlockSpec((BLK, N), lambda i: (i, 0)),
            pl.BlockSpec((N, N), lambda i: (0, 0)),
            pl.BlockSpec((N, H), lambda i: (0, 0)),
            pl.BlockSpec((BLK, H), lambda i: (i, 0)),
            pl.BlockSpec((BLK, 128), lambda i: (i, 0)),
            pl.BlockSpec((BLK, H), lambda i: (i, 0)),
            pl.BlockSpec((2 * H, H), lambda i: (0, 0)),
            pl.BlockSpec((1, H), lambda i: (0, 0)),
            pl.BlockSpec((H, H), lambda i: (0, 0)),
            pl.BlockSpec((1, H), lambda i: (0, 0)),
            pl.BlockSpec((3 * H, H), lambda i: (0, 0)),
            pl.BlockSpec((1, H), lambda i: (0, 0)),
            pl.BlockSpec((H, H), lambda i: (0, 0)),
            pl.BlockSpec((1, H), lambda i: (0, 0)),
        ],
        out_specs=[
            pl.BlockSpec((BLK, H), lambda i: (i, 0)),
            pl.BlockSpec((BLK, H), lambda i: (i, 0)),
        ],
        out_shape=[
            jax.ShapeDtypeStruct((N, H), jnp.float32),
            jax.ShapeDtypeStruct((N, H), jnp.float32),
        ],
    )(Ab, Ab, nt, nt, deg, nsum, Ws1, bs1, Ws2, bs2, Wh1, bh1, Wh2, bh2)


def kernel(x, edge_index, Wn, bn, We, be, Ws1, bs1, Ws2, bs2, Wh1, bh1, Wh2, bh2):
    row = edge_index[0]
    col = edge_index[1]

    nt, P, Q, flat2 = _tc_prep(x, Wn, bn.reshape(1, H), We, be.reshape(1, H),
                               row.reshape(EROWS, 128), col.reshape(EROWS, 128))

    Ab = _sc_adj(flat2).reshape(N, N).astype(jnp.bfloat16)

    edge_tokens = _sc_edges(P, Q, row.reshape(CHUNKS, CH),
                            col.reshape(CHUNKS, CH))

    nsum, deg = _tc_agg(row.reshape(ESTEPS, 1, ECH),
                        col.reshape(ESTEPS, 1, ECH),
                        nt.astype(jnp.bfloat16))

    sub, nb = _tc_main(Ab, nt, deg, nsum, Ws1, bs1.reshape(1, H),
                       Ws2, bs2.reshape(1, H), Wh1, bh1.reshape(1, H),
                       Wh2, bh2.reshape(1, H))
    return nt, edge_tokens, sub, nb
